# baseline (speedup 1.0000x reference)
import functools

import jax
import jax.numpy as jnp
from jax import lax
from jax.experimental import pallas as pl
from jax.experimental.pallas import tpu as pltpu
from jax.experimental.pallas import tpu_sc as plsc

N = 10000
H = 128
NC = 2
NS = 16
NW = NC * NS
LANES = 16
NPAD = 10240
RPT = NPAD // NW
DCH = NPAD // NS
DUMMY = N
ECH = 128
TVR = 1008
BLK = 512


def _sc_mesh():
    return plsc.VectorSubcoreMesh(core_axis_name="c", subcore_axis_name="s",
                                  num_cores=NC, num_subcores=NS)


def _sc_deg(pk):
    nec = pk.shape[0] // NW

    @functools.partial(
        pl.kernel,
        out_type=jax.ShapeDtypeStruct((NC, NPAD, H), jnp.float32),
        mesh=_sc_mesh(),
        scratch_types=[
            pltpu.VMEM((nec,), jnp.int32),
            pltpu.VMEM((nec, ECH), jnp.int32),
            pltpu.VMEM((ECH, H), jnp.float32),
            pltpu.VMEM_SHARED((NPAD, H), jnp.float32),
            pltpu.SemaphoreType.DMA,
        ],
    )
    def k(pk_h, degp_h, idxb, colv, onesb, acc, sem):
        c = lax.axis_index("c")
        s = lax.axis_index("s")
        t = c * NS + s
        for kk in range(nec // LANES):
            idxb[pl.ds(kk * LANES, LANES)] = (
                lax.iota(jnp.int32, LANES) + (t * nec + kk * LANES))
        pltpu.async_copy(pk_h.at[idxb], colv, sem).wait()

        def ub(i, _):
            j = i // (ECH // LANES)
            kk = i % (ECH // LANES)
            v = colv[j, pl.ds(kk * LANES, LANES)]
            colv[j, pl.ds(kk * LANES, LANES)] = jnp.bitwise_and(v, 16383)
            return 0
        lax.fori_loop(0, nec * (ECH // LANES), ub, 0)

        def zb(i, _):
            for kk in range(H // LANES):
                onesb[i, pl.ds(kk * LANES, LANES)] = (
                    jnp.zeros((LANES,), jnp.float32))
            return 0
        lax.fori_loop(0, ECH, zb, 0)
        for z in range(DCH // ECH):
            pltpu.sync_copy(onesb, acc.at[pl.ds(s * DCH + z * ECH, ECH)])

        def ob(i, _):
            for kk in range(H // LANES):
                onesb[i, pl.ds(kk * LANES, LANES)] = (
                    jnp.ones((LANES,), jnp.float32))
            return 0
        lax.fori_loop(0, ECH, ob, 0)
        plsc.subcore_barrier()

        def dbody(j, _):
            pltpu.sync_copy(onesb, acc.at[colv.at[j]], add=True)
            return 0
        lax.fori_loop(0, nec, dbody, 0)
        plsc.subcore_barrier()
        pltpu.sync_copy(acc.at[pl.ds(s * DCH, DCH)],
                        degp_h.at[c, pl.ds(s * DCH, DCH)])

    return k(pk)


def _sc_gather_rows(t0, xp):
    @functools.partial(
        pl.kernel,
        out_type=jax.ShapeDtypeStruct((NPAD, H), jnp.float32),
        mesh=_sc_mesh(),
        scratch_types=[
            pltpu.VMEM((4, 80), jnp.int32),
            pltpu.VMEM((RPT, H), jnp.float32),
            pltpu.SemaphoreType.DMA,
        ],
    )
    def k(t0_h, xp_h, hw0_h, xv, gbuf, sem):
        c = lax.axis_index("c")
        s = lax.axis_index("s")
        t = c * NS + s
        pltpu.sync_copy(xp_h.at[t], xv)
        for j in range(4):
            pltpu.async_copy(t0_h.at[xv.at[j]],
                             gbuf.at[pl.ds(j * 80, 80)], sem).wait()
        pltpu.sync_copy(gbuf, hw0_h.at[pl.ds(t * RPT, RPT)])

    return k(t0, xp)


def _sc_edge(g, pk):
    nec = pk.shape[0] // NW

    assert nec % 2 == 0

    @functools.partial(
        pl.kernel,
        out_type=jax.ShapeDtypeStruct((NC, NPAD, H), jnp.float32),
        mesh=_sc_mesh(),
        scratch_types=[
            pltpu.VMEM((nec,), jnp.int32),
            pltpu.VMEM((nec, ECH), jnp.int32),
            pltpu.VMEM((2, ECH), jnp.int32),
            pltpu.VMEM((2, ECH), jnp.int32),
            pltpu.VMEM((ECH, H), jnp.float32),
            pltpu.VMEM((ECH, H), jnp.float32),
            pltpu.VMEM_SHARED((NPAD, H), jnp.float32),
            pltpu.SemaphoreType.DMA,
            pltpu.SemaphoreType.DMA,
        ],
    )
    def k(g_h, pk_h, out_h, idxb, pkv, rowc, colc, bufa, bufb, acc,
          sema, semb):
        c = lax.axis_index("c")
        s = lax.axis_index("s")
        t = c * NS + s
        for kk in range(nec // LANES):
            idxb[pl.ds(kk * LANES, LANES)] = (
                lax.iota(jnp.int32, LANES) + (t * nec + kk * LANES))
        pltpu.async_copy(pk_h.at[idxb], pkv, sema).wait()

        def unpack(b, j):
            for kk in range(ECH // LANES):
                v = pkv[j, pl.ds(kk * LANES, LANES)]
                rowc[b, pl.ds(kk * LANES, LANES)] = jnp.right_shift(v, 14)
                colc[b, pl.ds(kk * LANES, LANES)] = jnp.bitwise_and(v, 16383)

        def zb(i, _):
            for kk in range(H // LANES):
                bufa[i, pl.ds(kk * LANES, LANES)] = (
                    jnp.zeros((LANES,), jnp.float32))
            return 0
        lax.fori_loop(0, ECH, zb, 0)
        for z in range(DCH // ECH):
            pltpu.sync_copy(bufa, acc.at[pl.ds(s * DCH + z * ECH, ECH)])
        plsc.subcore_barrier()

        def fire(buf, sem, b):
            pltpu.async_copy(g_h.at[rowc.at[b]], buf, sem)

        def drain(buf, sem):
            pltpu.make_async_copy(g_h.at[rowc.at[0]], buf, sem).wait()

        unpack(0, 0)
        fire(bufa, sema, 0)

        def outer(p, _):
            j0 = 2 * p
            unpack(1, j0 + 1)
            drain(bufa, sema)
            fire(bufb, semb, 1)
            pltpu.sync_copy(bufa, acc.at[colc.at[0]], add=True)

            @pl.when(p < nec // 2 - 1)
            def _():
                unpack(0, j0 + 2)
                drain(bufb, semb)
                fire(bufa, sema, 0)

            @pl.when(p == nec // 2 - 1)
            def _():
                drain(bufb, semb)
            pltpu.sync_copy(bufb, acc.at[colc.at[1]], add=True)
            return 0
        lax.fori_loop(0, nec // 2, outer, 0)
        plsc.subcore_barrier()
        pltpu.sync_copy(acc.at[pl.ds(s * DCH, DCH)],
                        out_h.at[c, pl.ds(s * DCH, DCH)])

    return k(g, pk)


def _sc_pool(h3, sgp):
    @functools.partial(
        pl.kernel,
        out_type=jax.ShapeDtypeStruct((512, H), jnp.float32),
        mesh=_sc_mesh(),
        scratch_types=[
            pltpu.VMEM((16, 64), jnp.int32),
            pltpu.VMEM((64,), jnp.int32),
            pltpu.VMEM((64, H), jnp.float32),
            pltpu.VMEM((16, H), jnp.float32),
            pltpu.SemaphoreType.DMA,
        ],
    )
    def k(h3_h, sg_h, out_h, sgv, idxb, rbuf, sumb, sem):
        c = lax.axis_index("c")
        s = lax.axis_index("s")
        t = c * NS + s
        pltpu.sync_copy(sg_h.at[t], sgv)

        def qbody(q, _):
            for kk in range(4):
                v = sgv[q, pl.ds(kk * LANES, LANES)]
                idxb[pl.ds(kk * LANES, LANES)] = jnp.where(v < 0, DUMMY, v)
            pltpu.async_copy(h3_h.at[idxb], rbuf, sem).wait()
            for k2 in range(H // LANES):
                def sbody(jj, acc):
                    return acc + rbuf[jj, pl.ds(k2 * LANES, LANES)]
                acc = lax.fori_loop(0, 64, sbody,
                                    jnp.zeros((LANES,), jnp.float32))
                sumb[q, pl.ds(k2 * LANES, LANES)] = acc
            return 0
        lax.fori_loop(0, 16, qbody, 0)
        pltpu.sync_copy(sumb, out_h.at[pl.ds(t * 16, 16)])

    return k(h3, sgp)


def _tc_table(embp, w0):
    def body(a_ref, b_ref, o_ref):
        o_ref[...] = jnp.dot(a_ref[...], b_ref[...],
                             preferred_element_type=jnp.float32)
    return pl.pallas_call(
        body, out_shape=jax.ShapeDtypeStruct((TVR, H), jnp.float32))(embp, w0)


def _tc_dinv(degp3):
    def body(d_ref, o_ref):
        sfull = d_ref[0] + d_ref[1] + 1.0
        ridx = lax.broadcasted_iota(jnp.int32, (NPAD // 128, 128), 0) * 128 + \
            lax.broadcasted_iota(jnp.int32, (NPAD // 128, 128), 1)
        o_ref[...] = jnp.where(ridx < N, lax.rsqrt(sfull), 0.0)
    return pl.pallas_call(
        body,
        out_shape=jax.ShapeDtypeStruct((NPAD // 128, 128), jnp.float32))(degp3)


def _tc_scale(dinvc, hw0):
    def body(d_ref, h_ref, o_ref):
        o_ref[...] = d_ref[...] * h_ref[...]
    return pl.pallas_call(
        body,
        grid=(NPAD // BLK,),
        in_specs=[pl.BlockSpec((BLK, 1), lambda i: (i, 0)),
                  pl.BlockSpec((BLK, H), lambda i: (i, 0))],
        out_specs=pl.BlockSpec((BLK, H), lambda i: (i, 0)),
        out_shape=jax.ShapeDtypeStruct((NPAD, H), jnp.float32))(dinvc, hw0)


def _tc_layer(dinvc, s0, s1, g, b, w):
    def body(d_ref, s0_ref, s1_ref, g_ref, b_ref, w_ref, o_ref):
        d = d_ref[...]
        h = jnp.maximum(d * (s0_ref[...] + s1_ref[...] + g_ref[...])
                        + b_ref[...], 0.0)
        o_ref[...] = d * jnp.dot(h, w_ref[...],
                                 preferred_element_type=jnp.float32)
    return pl.pallas_call(
        body,
        grid=(NPAD // BLK,),
        in_specs=[pl.BlockSpec((BLK, 1), lambda i: (i, 0)),
                  pl.BlockSpec((BLK, H), lambda i: (i, 0)),
                  pl.BlockSpec((BLK, H), lambda i: (i, 0)),
                  pl.BlockSpec((BLK, H), lambda i: (i, 0)),
                  pl.BlockSpec((1, H), lambda i: (0, 0)),
                  pl.BlockSpec((H, H), lambda i: (0, 0))],
        out_specs=pl.BlockSpec((BLK, H), lambda i: (i, 0)),
        out_shape=jax.ShapeDtypeStruct((NPAD, H), jnp.float32))(
            dinvc, s0, s1, g, b, w)


def _tc_final(dinvc, s0, s1, g, b):
    def body(d_ref, s0_ref, s1_ref, g_ref, b_ref, o_ref):
        i = pl.program_id(0)
        h = jnp.maximum(d_ref[...] * (s0_ref[...] + s1_ref[...] + g_ref[...])
                        + b_ref[...], 0.0)
        rows = i * BLK + lax.broadcasted_iota(jnp.int32, (BLK, H), 0)
        o_ref[...] = jnp.where(rows < N, h, 0.0)
    return pl.pallas_call(
        body,
        grid=(NPAD // BLK,),
        in_specs=[pl.BlockSpec((BLK, 1), lambda i: (i, 0)),
                  pl.BlockSpec((BLK, H), lambda i: (i, 0)),
                  pl.BlockSpec((BLK, H), lambda i: (i, 0)),
                  pl.BlockSpec((BLK, H), lambda i: (i, 0)),
                  pl.BlockSpec((1, H), lambda i: (0, 0))],
        out_specs=pl.BlockSpec((BLK, H), lambda i: (i, 0)),
        out_shape=jax.ShapeDtypeStruct((NPAD, H), jnp.float32))(
            dinvc, s0, s1, g, b)


def _tc_head(sums, sg, w1, b1, w2, b2):
    odim = w2.shape[1]

    def body(s_ref, sg_ref, w1_ref, b1_ref, w2_ref, b2_ref, o_ref):
        cnt = jnp.sum((sg_ref[...] != -1).astype(jnp.float32), axis=1,
                      keepdims=True)
        pooled = s_ref[...] / jnp.maximum(cnt, 1.0)
        hid = jnp.maximum(jnp.dot(pooled, w1_ref[...],
                                  preferred_element_type=jnp.float32)
                          + b1_ref[...], 0.0)
        o_ref[...] = jnp.dot(hid, w2_ref[...],
                             preferred_element_type=jnp.float32) + b2_ref[...]
    return pl.pallas_call(
        body, out_shape=jax.ShapeDtypeStruct((sums.shape[0], odim),
                                             jnp.float32))(
            sums, sg, w1, b1, w2, b2)


@jax.jit
def kernel(x, edge_index, edge_attr, subg_nodes, embedding, convW, convB,
           mlpW1, mlpB1, mlpW2, mlpB2):
    x = x.astype(jnp.int32)
    ei = edge_index.astype(jnp.int32)
    e = ei.shape[1]
    nec = -(-e // (NW * ECH))
    nec = -(-nec // 4) * 4
    etot = NW * nec * ECH
    rowf = jnp.concatenate([ei[0], jnp.zeros((etot - e,), jnp.int32)])
    colf = jnp.concatenate([ei[1], jnp.full((etot - e,), DUMMY, jnp.int32)])
    pk = ((rowf << 14) | colf).reshape(NW * nec, ECH)
    xp = jnp.concatenate(
        [x, jnp.zeros((NPAD - N,), jnp.int32)]).reshape(NW, 4, 80)
    embp = jnp.pad(embedding, ((0, TVR - embedding.shape[0]), (0, 0)))
    sg = subg_nodes.astype(jnp.int32)
    sgp = sg.reshape(NW, 512 // NW, 64)

    t0 = _tc_table(embp, convW[0])
    degp = _sc_deg(pk)
    hw0 = _sc_gather_rows(t0, xp)
    dinv2d = _tc_dinv(degp[:, :, 0].reshape(NC, NPAD // 128, 128))
    dinvc = dinv2d.reshape(NPAD, 1)
    g = _tc_scale(dinvc, hw0)
    h3 = None
    for i in range(convW.shape[0]):
        sp = _sc_edge(g, pk)
        if i < convW.shape[0] - 1:
            g = _tc_layer(dinvc, sp[0], sp[1], g, convB[i].reshape(1, H),
                          convW[i + 1])
        else:
            h3 = _tc_final(dinvc, sp[0], sp[1], g, convB[i].reshape(1, H))
    sums = _sc_pool(h3, sgp)
    return _tc_head(sums, sg, mlpW1, mlpB1.reshape(1, H), mlpW2,
                    mlpB2.reshape(1, -1))


# speedup vs baseline: 1.5007x; 1.5007x over previous
import functools

import jax
import jax.numpy as jnp
from jax import lax
from jax.experimental import pallas as pl
from jax.experimental.pallas import tpu as pltpu
from jax.experimental.pallas import tpu_sc as plsc

N = 10000
H = 128
NC = 2
NS = 16
NW = NC * NS
LANES = 16
NPAD = 10240
RPT = NPAD // NW
DCH = NPAD // NS
DUMMY = N
ECH = 128
TVR = 1008
BLK = 512


def _sc_mesh():
    return plsc.VectorSubcoreMesh(core_axis_name="c", subcore_axis_name="s",
                                  num_cores=NC, num_subcores=NS)


def _sc_deg(colp):
    nec = colp.shape[1]

    @functools.partial(
        pl.kernel,
        out_type=jax.ShapeDtypeStruct((NC, NPAD, H), jnp.float32),
        mesh=_sc_mesh(),
        scratch_types=[
            pltpu.VMEM((nec, ECH), jnp.int32),
            pltpu.VMEM((ECH, H), jnp.float32),
            pltpu.VMEM_SHARED((NPAD, H), jnp.float32),
            pltpu.SemaphoreType.DMA,
        ],
    )
    def k(colp_h, degp_h, colv, onesb, acc, sem):
        c = lax.axis_index("c")
        s = lax.axis_index("s")
        t = c * NS + s
        pltpu.sync_copy(colp_h.at[t], colv)

        def zb(i, _):
            for kk in range(H // LANES):
                onesb[i, pl.ds(kk * LANES, LANES)] = (
                    jnp.zeros((LANES,), jnp.float32))
            return 0
        lax.fori_loop(0, ECH, zb, 0)
        for z in range(DCH // ECH):
            pltpu.sync_copy(onesb, acc.at[pl.ds(s * DCH + z * ECH, ECH)])

        def ob(i, _):
            for kk in range(H // LANES):
                onesb[i, pl.ds(kk * LANES, LANES)] = (
                    jnp.ones((LANES,), jnp.float32))
            return 0
        lax.fori_loop(0, ECH, ob, 0)
        plsc.subcore_barrier()

        def dbody(j, _):
            pltpu.sync_copy(onesb, acc.at[colv.at[j]], add=True)
            return 0
        lax.fori_loop(0, nec, dbody, 0)
        plsc.subcore_barrier()
        pltpu.sync_copy(acc.at[pl.ds(s * DCH, DCH)],
                        degp_h.at[c, pl.ds(s * DCH, DCH)])

    return k(colp)


def _sc_gather_rows(t0, xp):
    @functools.partial(
        pl.kernel,
        out_type=jax.ShapeDtypeStruct((NPAD, H), jnp.float32),
        mesh=_sc_mesh(),
        scratch_types=[
            pltpu.VMEM((4, 80), jnp.int32),
            pltpu.VMEM((RPT, H), jnp.float32),
            pltpu.SemaphoreType.DMA,
        ],
    )
    def k(t0_h, xp_h, hw0_h, xv, gbuf, sem):
        c = lax.axis_index("c")
        s = lax.axis_index("s")
        t = c * NS + s
        pltpu.sync_copy(xp_h.at[t], xv)
        for j in range(4):
            pltpu.async_copy(t0_h.at[xv.at[j]],
                             gbuf.at[pl.ds(j * 80, 80)], sem).wait()
        pltpu.sync_copy(gbuf, hw0_h.at[pl.ds(t * RPT, RPT)])

    return k(t0, xp)


def _sc_edge(g, rowp, colp):
    nec = rowp.shape[1]

    @functools.partial(
        pl.kernel,
        out_type=jax.ShapeDtypeStruct((NC, NPAD, H), jnp.float32),
        mesh=_sc_mesh(),
        scratch_types=[
            pltpu.VMEM((nec, ECH), jnp.int32),
            pltpu.VMEM((nec, ECH), jnp.int32),
            pltpu.VMEM((ECH, H), jnp.float32),
            pltpu.VMEM_SHARED((NPAD, H), jnp.float32),
            pltpu.SemaphoreType.DMA,
        ],
    )
    def k(g_h, rowp_h, colp_h, out_h, rowi, colv, buf, acc, sem):
        c = lax.axis_index("c")
        s = lax.axis_index("s")
        t = c * NS + s
        pltpu.sync_copy(rowp_h.at[t], rowi)
        pltpu.sync_copy(colp_h.at[t], colv)

        def zb(i, _):
            for kk in range(H // LANES):
                buf[i, pl.ds(kk * LANES, LANES)] = (
                    jnp.zeros((LANES,), jnp.float32))
            return 0
        lax.fori_loop(0, ECH, zb, 0)
        for z in range(DCH // ECH):
            pltpu.sync_copy(buf, acc.at[pl.ds(s * DCH + z * ECH, ECH)])
        plsc.subcore_barrier()

        def ebody(j, _):
            pltpu.async_copy(g_h.at[rowi.at[j]], buf, sem).wait()
            pltpu.sync_copy(buf, acc.at[colv.at[j]], add=True)
            return 0
        lax.fori_loop(0, nec, ebody, 0)
        plsc.subcore_barrier()
        pltpu.sync_copy(acc.at[pl.ds(s * DCH, DCH)],
                        out_h.at[c, pl.ds(s * DCH, DCH)])

    return k(g, rowp, colp)


def _sc_pool(h3, sgp):
    @functools.partial(
        pl.kernel,
        out_type=jax.ShapeDtypeStruct((512, H), jnp.float32),
        mesh=_sc_mesh(),
        scratch_types=[
            pltpu.VMEM((16, 64), jnp.int32),
            pltpu.VMEM((2, 64), jnp.int32),
            pltpu.VMEM((2, 64, H), jnp.float32),
            pltpu.VMEM((16, H), jnp.float32),
            pltpu.SemaphoreType.DMA,
            pltpu.SemaphoreType.DMA,
        ],
    )
    def k(h3_h, sg_h, out_h, sgv, idxb, rbuf, sumb, sema, semb):
        c = lax.axis_index("c")
        s = lax.axis_index("s")
        t = c * NS + s
        pltpu.sync_copy(sg_h.at[t], sgv)
        sems = (sema, semb)

        def fire(q, b):
            for kk in range(4):
                v = sgv[q, pl.ds(kk * LANES, LANES)]
                idxb[b, pl.ds(kk * LANES, LANES)] = jnp.where(v < 0, DUMMY, v)
            pltpu.async_copy(h3_h.at[idxb.at[b]], rbuf.at[b], sems[b])

        def drain(b):
            pltpu.make_async_copy(h3_h.at[idxb.at[0]], rbuf.at[b],
                                  sems[b]).wait()

        def accum(q, b):
            def sbody(jj, accs):
                return tuple(
                    accs[k2] + rbuf[b, jj, pl.ds(k2 * LANES, LANES)]
                    for k2 in range(H // LANES))
            accs = lax.fori_loop(
                0, 64, sbody,
                tuple(jnp.zeros((LANES,), jnp.float32)
                      for _ in range(H // LANES)))
            for k2 in range(H // LANES):
                sumb[q, pl.ds(k2 * LANES, LANES)] = accs[k2]

        fire(0, 0)

        def qbody(p, _):
            q0 = 2 * p
            drain(0)
            fire(q0 + 1, 1)
            accum(q0, 0)
            drain(1)

            @pl.when(p < 7)
            def _():
                fire(q0 + 2, 0)
            accum(q0 + 1, 1)
            return 0
        lax.fori_loop(0, 8, qbody, 0)
        pltpu.sync_copy(sumb, out_h.at[pl.ds(t * 16, 16)])

    return k(h3, sgp)


def _tc_table(embp, w0):
    def body(a_ref, b_ref, o_ref):
        o_ref[...] = jnp.dot(a_ref[...], b_ref[...],
                             preferred_element_type=jnp.float32)
    return pl.pallas_call(
        body, out_shape=jax.ShapeDtypeStruct((TVR, H), jnp.float32))(embp, w0)


def _tc_dinv(degp3):
    def body(d_ref, o_ref):
        sfull = d_ref[0] + d_ref[1] + 1.0
        ridx = lax.broadcasted_iota(jnp.int32, (NPAD // 128, 128), 0) * 128 + \
            lax.broadcasted_iota(jnp.int32, (NPAD // 128, 128), 1)
        o_ref[...] = jnp.where(ridx < N, lax.rsqrt(sfull), 0.0)
    return pl.pallas_call(
        body,
        out_shape=jax.ShapeDtypeStruct((NPAD // 128, 128), jnp.float32))(degp3)


def _tc_scale(dinvc, hw0):
    def body(d_ref, h_ref, o_ref):
        o_ref[...] = d_ref[...] * h_ref[...]
    return pl.pallas_call(
        body,
        grid=(NPAD // BLK,),
        in_specs=[pl.BlockSpec((BLK, 1), lambda i: (i, 0)),
                  pl.BlockSpec((BLK, H), lambda i: (i, 0))],
        out_specs=pl.BlockSpec((BLK, H), lambda i: (i, 0)),
        out_shape=jax.ShapeDtypeStruct((NPAD, H), jnp.float32))(dinvc, hw0)


def _tc_layer(dinvc, sp, g, b, w):
    def body(d_ref, s0_ref, s1_ref, g_ref, b_ref, w_ref, o_ref):
        d = d_ref[...]
        h = jnp.maximum(d * (s0_ref[0] + s1_ref[0] + g_ref[...])
                        + b_ref[...], 0.0)
        o_ref[...] = d * jnp.dot(h, w_ref[...],
                                 preferred_element_type=jnp.float32)
    return pl.pallas_call(
        body,
        grid=(NPAD // BLK,),
        in_specs=[pl.BlockSpec((BLK, 1), lambda i: (i, 0)),
                  pl.BlockSpec((1, BLK, H), lambda i: (0, i, 0)),
                  pl.BlockSpec((1, BLK, H), lambda i: (1, i, 0)),
                  pl.BlockSpec((BLK, H), lambda i: (i, 0)),
                  pl.BlockSpec((1, H), lambda i: (0, 0)),
                  pl.BlockSpec((H, H), lambda i: (0, 0))],
        out_specs=pl.BlockSpec((BLK, H), lambda i: (i, 0)),
        out_shape=jax.ShapeDtypeStruct((NPAD, H), jnp.float32))(
            dinvc, sp, sp, g, b, w)


def _tc_final(dinvc, sp, g, b):
    def body(d_ref, s0_ref, s1_ref, g_ref, b_ref, o_ref):
        i = pl.program_id(0)
        h = jnp.maximum(d_ref[...] * (s0_ref[0] + s1_ref[0] + g_ref[...])
                        + b_ref[...], 0.0)
        rows = i * BLK + lax.broadcasted_iota(jnp.int32, (BLK, H), 0)
        o_ref[...] = jnp.where(rows < N, h, 0.0)
    return pl.pallas_call(
        body,
        grid=(NPAD // BLK,),
        in_specs=[pl.BlockSpec((BLK, 1), lambda i: (i, 0)),
                  pl.BlockSpec((1, BLK, H), lambda i: (0, i, 0)),
                  pl.BlockSpec((1, BLK, H), lambda i: (1, i, 0)),
                  pl.BlockSpec((BLK, H), lambda i: (i, 0)),
                  pl.BlockSpec((1, H), lambda i: (0, 0))],
        out_specs=pl.BlockSpec((BLK, H), lambda i: (i, 0)),
        out_shape=jax.ShapeDtypeStruct((NPAD, H), jnp.float32))(
            dinvc, sp, sp, g, b)


def _tc_head(sums, sg, w1, b1, w2, b2):
    odim = w2.shape[1]

    def body(s_ref, sg_ref, w1_ref, b1_ref, w2_ref, b2_ref, o_ref):
        cnt = jnp.sum((sg_ref[...] != -1).astype(jnp.float32), axis=1,
                      keepdims=True)
        pooled = s_ref[...] / jnp.maximum(cnt, 1.0)
        hid = jnp.maximum(jnp.dot(pooled, w1_ref[...],
                                  preferred_element_type=jnp.float32)
                          + b1_ref[...], 0.0)
        o_ref[...] = jnp.dot(hid, w2_ref[...],
                             preferred_element_type=jnp.float32) + b2_ref[...]
    return pl.pallas_call(
        body, out_shape=jax.ShapeDtypeStruct((sums.shape[0], odim),
                                             jnp.float32))(
            sums, sg, w1, b1, w2, b2)


@jax.jit
def kernel(x, edge_index, edge_attr, subg_nodes, embedding, convW, convB,
           mlpW1, mlpB1, mlpW2, mlpB2):
    x = x.astype(jnp.int32)
    ei = edge_index.astype(jnp.int32)
    e = ei.shape[1]
    nec = -(-e // (NW * ECH))
    etot = NW * nec * ECH
    rowp = jnp.concatenate(
        [ei[0], jnp.zeros((etot - e,), jnp.int32)]).reshape(NW, nec, ECH)
    colp = jnp.concatenate(
        [ei[1], jnp.full((etot - e,), DUMMY, jnp.int32)]).reshape(NW, nec, ECH)
    xp = jnp.concatenate(
        [x, jnp.zeros((NPAD - N,), jnp.int32)]).reshape(NW, 4, 80)
    embp = jnp.pad(embedding, ((0, TVR - embedding.shape[0]), (0, 0)))
    sg = subg_nodes.astype(jnp.int32)
    sgp = sg.reshape(NW, 512 // NW, 64)

    t0 = _tc_table(embp, convW[0])
    degp = _sc_deg(colp)
    hw0 = _sc_gather_rows(t0, xp)
    dinv2d = _tc_dinv(degp[:, :, 0].reshape(NC, NPAD // 128, 128))
    dinvc = dinv2d.reshape(NPAD, 1)
    g = _tc_scale(dinvc, hw0)
    h3 = None
    for i in range(convW.shape[0]):
        sp = _sc_edge(g, rowp, colp)
        if i < convW.shape[0] - 1:
            g = _tc_layer(dinvc, sp, g, convB[i].reshape(1, H), convW[i + 1])
        else:
            h3 = _tc_final(dinvc, sp, g, convB[i].reshape(1, H))
    sums = _sc_pool(h3, sgp)
    return _tc_head(sums, sg, mlpW1, mlpB1.reshape(1, H), mlpW2,
                    mlpB2.reshape(1, -1))
